# baseline (speedup 1.0000x reference)
import functools

import jax
import jax.numpy as jnp
from jax import lax
from jax.experimental import pallas as pl
from jax.experimental.pallas import tpu as pltpu
from jax.experimental.pallas import tpu_sc as plsc

NC = 2
NS = 16
NW = NC * NS
VOCAB = 1000
D = 1000
D_PAD = 1024
B_TOTAL = 1024 * 50
B_PER_W = B_TOTAL // NW
CHUNK = 40
N_CHUNKS = B_PER_W // CHUNK

_mesh = plsc.VectorSubcoreMesh(core_axis_name="c", subcore_axis_name="s")


@jax.jit
def _gather(table_pad, idx3):
    @functools.partial(
        pl.kernel,
        mesh=_mesh,
        out_type=jax.ShapeDtypeStruct((B_TOTAL, D_PAD), jnp.float32),
        scratch_types=[
            pltpu.VMEM((N_CHUNKS, CHUNK), jnp.int32),
            pltpu.VMEM((CHUNK, D_PAD), jnp.float32),
            pltpu.VMEM((CHUNK, D_PAD), jnp.float32),
            pltpu.SemaphoreType.DMA,
            pltpu.SemaphoreType.DMA,
        ],
    )
    def k(table_hbm, idx_hbm, out_hbm, idx_v, rows0, rows1, sem0, sem1):
        wid = lax.axis_index("s") * NC + lax.axis_index("c")
        pltpu.sync_copy(idx_hbm.at[wid], idx_v)
        base = wid * B_PER_W

        pltpu.async_copy(table_hbm.at[idx_v.at[0]], rows0, sem0)

        def wait_gather(buf, sem):
            pltpu.make_async_copy(table_hbm.at[pl.ds(0, CHUNK)], buf, sem).wait()

        @pl.loop(0, N_CHUNKS, step=2)
        def _(c):
            pltpu.async_copy(table_hbm.at[idx_v.at[c + 1]], rows1, sem1)
            wait_gather(rows0, sem0)
            pltpu.sync_copy(rows0, out_hbm.at[pl.ds(base + c * CHUNK, CHUNK)])

            @pl.when(c + 2 < N_CHUNKS)
            def _():
                pltpu.async_copy(table_hbm.at[idx_v.at[c + 2]], rows0, sem0)

            wait_gather(rows1, sem1)
            pltpu.sync_copy(
                rows1, out_hbm.at[pl.ds(base + (c + 1) * CHUNK, CHUNK)]
            )

    return k(table_pad, idx3)


BATCH = 1024
SEQ = 50
BB = 8
_N_DESPAD = BATCH // BB


def _despad_body(i_ref, o_ref):
    o_ref[...] = i_ref[:, :D].reshape(BB, SEQ, D)


_despad = pl.pallas_call(
    _despad_body,
    grid=(_N_DESPAD,),
    in_specs=[pl.BlockSpec((BB * SEQ, D_PAD), lambda i: (i, 0))],
    out_specs=pl.BlockSpec((BB, SEQ, D), lambda i: (i, 0, 0)),
    out_shape=jax.ShapeDtypeStruct((BATCH, SEQ, D), jnp.float32),
)


def kernel(x, table):
    table_pad = jnp.pad(table, ((0, 0), (0, D_PAD - D)))
    idx3 = x.reshape(NW, N_CHUNKS, CHUNK)
    out_pad = _gather(table_pad, idx3)
    return _despad(out_pad)


# speedup vs baseline: 1.4101x; 1.4101x over previous
import functools

import jax
import jax.numpy as jnp
from jax import lax
from jax.experimental import pallas as pl
from jax.experimental.pallas import tpu as pltpu
from jax.experimental.pallas import tpu_sc as plsc

NC = 2
NS = 16
NW = NC * NS
VOCAB = 1000
D = 1000
D_PAD = 1024
BATCH = 1024
SEQ = 50
BBLK = 32
N_BBLK = BATCH // BBLK
N_UNITS = SEQ * N_BBLK
U_PER_W = N_UNITS // NW

_mesh = plsc.VectorSubcoreMesh(core_axis_name="c", subcore_axis_name="s")


@jax.jit
def _gather(table_pad, idxg):
    @functools.partial(
        pl.kernel,
        mesh=_mesh,
        out_type=jax.ShapeDtypeStruct((SEQ, BATCH, D_PAD), jnp.float32),
        scratch_types=[
            pltpu.VMEM((U_PER_W, BBLK), jnp.int32),
            pltpu.VMEM((BBLK, D_PAD), jnp.float32),
            pltpu.VMEM((BBLK, D_PAD), jnp.float32),
            pltpu.SemaphoreType.DMA,
            pltpu.SemaphoreType.DMA,
        ],
    )
    def k(table_hbm, idx_hbm, out_hbm, idx_v, rows0, rows1, sem0, sem1):
        wid = lax.axis_index("s") * NC + lax.axis_index("c")
        pltpu.sync_copy(idx_hbm.at[wid], idx_v)
        gbase = wid * U_PER_W

        def dst(u):
            g = gbase + u
            return out_hbm.at[g // N_BBLK, pl.ds((g % N_BBLK) * BBLK, BBLK)]

        pltpu.async_copy(table_hbm.at[idx_v.at[0]], rows0, sem0)

        def wait_gather(buf, sem):
            pltpu.make_async_copy(table_hbm.at[pl.ds(0, BBLK)], buf, sem).wait()

        @pl.loop(0, U_PER_W, step=2)
        def _(u):
            pltpu.async_copy(table_hbm.at[idx_v.at[u + 1]], rows1, sem1)
            wait_gather(rows0, sem0)
            pltpu.sync_copy(rows0, dst(u))

            @pl.when(u + 2 < U_PER_W)
            def _():
                pltpu.async_copy(table_hbm.at[idx_v.at[u + 2]], rows0, sem0)

            wait_gather(rows1, sem1)
            pltpu.sync_copy(rows1, dst(u + 1))

    return k(table_pad, idxg)


FB = 256
_N_F = D_PAD // FB


def _txp_body(i_ref, o_ref):
    o_ref[0] = jnp.swapaxes(i_ref[0], 0, 1)


_txp = pl.pallas_call(
    _txp_body,
    grid=(SEQ, _N_F),
    in_specs=[pl.BlockSpec((1, BATCH, FB), lambda s, j: (s, 0, j))],
    out_specs=pl.BlockSpec((1, FB, BATCH), lambda s, j: (s, j, 0)),
    out_shape=jax.ShapeDtypeStruct((SEQ, D, BATCH), jnp.float32),
)


def kernel(x, table):
    table_pad = jnp.pad(table, ((0, 0), (0, D_PAD - D)))
    idxg = x.T.reshape(NW, U_PER_W, BBLK)
    y_sm = _gather(table_pad, idxg)
    y_t = _txp(y_sm)
    return jnp.transpose(y_t, (2, 0, 1))


# speedup vs baseline: 1.6610x; 1.1780x over previous
import functools

import jax
import jax.numpy as jnp
from jax import lax
from jax.experimental import pallas as pl
from jax.experimental.pallas import tpu as pltpu
from jax.experimental.pallas import tpu_sc as plsc

NC = 2
NS = 16
NW = NC * NS
VOCAB = 1000
D = 1000
D_PAD = 1024
BATCH = 1024
SEQ = 50
BBLK = 32
N_BBLK = BATCH // BBLK
N_UNITS = SEQ * N_BBLK
U_PER_W = N_UNITS // NW

_mesh = plsc.VectorSubcoreMesh(core_axis_name="c", subcore_axis_name="s")


@jax.jit
def _gather(table_pad, idxg):
    @functools.partial(
        pl.kernel,
        mesh=_mesh,
        out_type=jax.ShapeDtypeStruct((SEQ, BATCH, D_PAD), jnp.float32),
        scratch_types=[
            pltpu.VMEM((U_PER_W, BBLK), jnp.int32),
            pltpu.VMEM((BBLK, D_PAD), jnp.float32),
            pltpu.VMEM((BBLK, D_PAD), jnp.float32),
            pltpu.SemaphoreType.DMA,
            pltpu.SemaphoreType.DMA,
        ],
    )
    def k(table_hbm, idx_hbm, out_hbm, idx_v, rows0, rows1, sem0, sem1):
        wid = lax.axis_index("s") * NC + lax.axis_index("c")
        pltpu.sync_copy(idx_hbm.at[wid], idx_v)
        gbase = wid * U_PER_W

        def dst(u):
            g = gbase + u
            return out_hbm.at[g // N_BBLK, pl.ds((g % N_BBLK) * BBLK, BBLK)]

        pltpu.async_copy(table_hbm.at[idx_v.at[0]], rows0, sem0)

        def wait_gather(buf, sem):
            pltpu.make_async_copy(table_hbm.at[pl.ds(0, BBLK)], buf, sem).wait()

        @pl.loop(0, U_PER_W, step=2)
        def _(u):
            pltpu.async_copy(table_hbm.at[idx_v.at[u + 1]], rows1, sem1)
            wait_gather(rows0, sem0)
            pltpu.sync_copy(rows0, dst(u))

            @pl.when(u + 2 < U_PER_W)
            def _():
                pltpu.async_copy(table_hbm.at[idx_v.at[u + 2]], rows0, sem0)

            wait_gather(rows1, sem1)
            pltpu.sync_copy(rows1, dst(u + 1))

    return k(table_pad, idxg)


FB = 512
_N_F = D_PAD // FB


def _txp_body(i_ref, o_ref):
    o_ref[0] = jnp.swapaxes(i_ref[0], 0, 1)


_txp = pl.pallas_call(
    _txp_body,
    grid=(SEQ, _N_F),
    in_specs=[pl.BlockSpec((1, BATCH, FB), lambda s, j: (s, 0, j))],
    out_specs=pl.BlockSpec((1, FB, BATCH), lambda s, j: (s, j, 0)),
    out_shape=jax.ShapeDtypeStruct((SEQ, D, BATCH), jnp.float32),
)


def kernel(x, table):
    table_pad = jnp.pad(table, ((0, 0), (0, D_PAD - D)))
    idxg = x.T.reshape(NW, U_PER_W, BBLK)
    y_sm = _gather(table_pad, idxg)
    y_t = _txp(y_sm)
    return jnp.transpose(y_t, (2, 0, 1))


# speedup vs baseline: 1.7423x; 1.0489x over previous
import functools

import jax
import jax.numpy as jnp
from jax import lax
from jax.experimental import pallas as pl
from jax.experimental.pallas import tpu as pltpu
from jax.experimental.pallas import tpu_sc as plsc

NC = 2
NS = 16
NW = NC * NS
VOCAB = 1000
D = 1000
D_PAD = 1024
BATCH = 1024
SEQ = 50
BBLK = 32
N_BBLK = BATCH // BBLK
NCH = 5
SCH = SEQ // NCH
N_UNITS = SCH * N_BBLK
U_PER_W = N_UNITS // NW

_mesh = plsc.VectorSubcoreMesh(core_axis_name="c", subcore_axis_name="s")


@jax.jit
def _gather(table_pad, idxg):
    @functools.partial(
        pl.kernel,
        mesh=_mesh,
        out_type=jax.ShapeDtypeStruct((SCH, BATCH, D_PAD), jnp.float32),
        scratch_types=[
            pltpu.VMEM((U_PER_W, BBLK), jnp.int32),
            pltpu.VMEM((BBLK, D_PAD), jnp.float32),
            pltpu.VMEM((BBLK, D_PAD), jnp.float32),
            pltpu.SemaphoreType.DMA,
            pltpu.SemaphoreType.DMA,
        ],
    )
    def k(table_hbm, idx_hbm, out_hbm, idx_v, rows0, rows1, sem0, sem1):
        wid = lax.axis_index("s") * NC + lax.axis_index("c")
        pltpu.sync_copy(idx_hbm.at[wid], idx_v)
        gbase = wid * U_PER_W

        def dst(u):
            g = gbase + u
            return out_hbm.at[g // N_BBLK, pl.ds((g % N_BBLK) * BBLK, BBLK)]

        pltpu.async_copy(table_hbm.at[idx_v.at[0]], rows0, sem0)

        def wait_gather(buf, sem):
            pltpu.make_async_copy(table_hbm.at[pl.ds(0, BBLK)], buf, sem).wait()

        @pl.loop(0, U_PER_W, step=2)
        def _(u):
            pltpu.async_copy(table_hbm.at[idx_v.at[u + 1]], rows1, sem1)
            wait_gather(rows0, sem0)
            pltpu.sync_copy(rows0, dst(u))

            @pl.when(u + 2 < U_PER_W)
            def _():
                pltpu.async_copy(table_hbm.at[idx_v.at[u + 2]], rows0, sem0)

            wait_gather(rows1, sem1)
            pltpu.sync_copy(rows1, dst(u + 1))

    return k(table_pad, idxg)


FB = 512
_N_F = D_PAD // FB


def _txp_body(i_ref, _a_ref, o_ref):
    o_ref[0] = jnp.swapaxes(i_ref[0], 0, 1)


def _txp0_body(i_ref, o_ref):
    o_ref[0] = jnp.swapaxes(i_ref[0], 0, 1)


def _make_txp(c):
    if c == 0:
        return pl.pallas_call(
            _txp0_body,
            grid=(SCH, _N_F),
            in_specs=[pl.BlockSpec((1, BATCH, FB), lambda s, j: (s, 0, j))],
            out_specs=pl.BlockSpec((1, FB, BATCH), lambda s, j: (s, j, 0)),
            out_shape=jax.ShapeDtypeStruct((SEQ, D, BATCH), jnp.float32),
        )
    return pl.pallas_call(
        _txp_body,
        grid=(SCH, _N_F),
        in_specs=[
            pl.BlockSpec((1, BATCH, FB), lambda s, j: (s, 0, j)),
            pl.BlockSpec(memory_space=pl.ANY),
        ],
        out_specs=pl.BlockSpec((1, FB, BATCH), lambda s, j, c=c: (c * SCH + s, j, 0)),
        out_shape=jax.ShapeDtypeStruct((SEQ, D, BATCH), jnp.float32),
        input_output_aliases={1: 0},
    )


_txps = [_make_txp(c) for c in range(NCH)]


def kernel(x, table):
    table_pad = jnp.pad(table, ((0, 0), (0, D_PAD - D)))
    xT = x.T
    chunks = [
        _gather(table_pad, xT[c * SCH:(c + 1) * SCH].reshape(NW, U_PER_W, BBLK))
        for c in range(NCH)
    ]
    out = _txps[0](chunks[0])
    for c in range(1, NCH):
        out = _txps[c](chunks[c], out)
    return jnp.transpose(out, (2, 0, 1))


# speedup vs baseline: 1.7619x; 1.0112x over previous
import functools

import jax
import jax.numpy as jnp
from jax import lax
from jax.experimental import pallas as pl
from jax.experimental.pallas import tpu as pltpu
from jax.experimental.pallas import tpu_sc as plsc

NC = 2
NS = 16
NW = NC * NS
VOCAB = 1000
D = 1000
D_PAD = 1024
BATCH = 1024
SEQ = 50
BBLK = 32
N_BBLK = BATCH // BBLK
NCH = 5
SCH = SEQ // NCH
N_UNITS = SCH * N_BBLK
U_PER_W = N_UNITS // NW

_mesh = plsc.VectorSubcoreMesh(core_axis_name="c", subcore_axis_name="s")


@jax.jit
def _gather(table_pad, idxg):
    @functools.partial(
        pl.kernel,
        mesh=_mesh,
        out_type=jax.ShapeDtypeStruct((SCH, BATCH, D_PAD), jnp.float32),
        scratch_types=[
            pltpu.VMEM((U_PER_W, BBLK), jnp.int32),
            pltpu.VMEM((BBLK, D_PAD), jnp.float32),
            pltpu.VMEM((BBLK, D_PAD), jnp.float32),
            pltpu.SemaphoreType.DMA,
            pltpu.SemaphoreType.DMA,
        ],
    )
    def k(table_hbm, idx_hbm, out_hbm, idx_v, rows0, rows1, sem0, sem1):
        wid = lax.axis_index("s") * NC + lax.axis_index("c")
        pltpu.sync_copy(idx_hbm.at[wid], idx_v)
        gbase = wid * U_PER_W

        def dst(u):
            g = gbase + u
            return out_hbm.at[g // N_BBLK, pl.ds((g % N_BBLK) * BBLK, BBLK)]

        pltpu.async_copy(table_hbm.at[idx_v.at[0]], rows0, sem0)

        def wait_gather(buf, sem):
            pltpu.make_async_copy(table_hbm.at[pl.ds(0, BBLK)], buf, sem).wait()

        @pl.loop(0, U_PER_W, step=2)
        def _(u):
            pltpu.async_copy(table_hbm.at[idx_v.at[u + 1]], rows1, sem1)
            wait_gather(rows0, sem0)
            pltpu.sync_copy(rows0, dst(u))

            @pl.when(u + 2 < U_PER_W)
            def _():
                pltpu.async_copy(table_hbm.at[idx_v.at[u + 2]], rows0, sem0)

            wait_gather(rows1, sem1)
            pltpu.sync_copy(rows1, dst(u + 1))

    return k(table_pad, idxg)


FB = 512
_N_F = D_PAD // FB


def _txp_body(i_ref, _a_ref, o_ref):
    o_ref[...] = jnp.swapaxes(i_ref[...], 1, 2)


def _txp0_body(i_ref, o_ref):
    o_ref[...] = jnp.swapaxes(i_ref[...], 1, 2)


def _make_txp(c):
    if c == 0:
        return pl.pallas_call(
            _txp0_body,
            grid=(SCH // 2, _N_F),
            in_specs=[pl.BlockSpec((2, BATCH, FB), lambda s, j: (s, 0, j))],
            out_specs=pl.BlockSpec((2, FB, BATCH), lambda s, j: (s, j, 0)),
            out_shape=jax.ShapeDtypeStruct((SEQ, D, BATCH), jnp.float32),
        )
    return pl.pallas_call(
        _txp_body,
        grid=(SCH // 2, _N_F),
        in_specs=[
            pl.BlockSpec((2, BATCH, FB), lambda s, j: (s, 0, j)),
            pl.BlockSpec(memory_space=pl.ANY),
        ],
        out_specs=pl.BlockSpec(
            (2, FB, BATCH), lambda s, j, c=c: (c * (SCH // 2) + s, j, 0)
        ),
        out_shape=jax.ShapeDtypeStruct((SEQ, D, BATCH), jnp.float32),
        input_output_aliases={1: 0},
    )


_txps = [_make_txp(c) for c in range(NCH)]


def kernel(x, table):
    table_pad = jnp.pad(table, ((0, 0), (0, D_PAD - D)))
    xT = x.T
    chunks = [
        _gather(table_pad, xT[c * SCH:(c + 1) * SCH].reshape(NW, U_PER_W, BBLK))
        for c in range(NCH)
    ]
    out = _txps[0](chunks[0])
    for c in range(1, NCH):
        out = _txps[c](chunks[c], out)
    return jnp.transpose(out, (2, 0, 1))
